# speedup vs baseline: 1.0004x; 1.0004x over previous
import functools

import jax
import jax.numpy as jnp
import numpy as np
from jax.experimental import pallas as pl
from jax.experimental.pallas import tpu as pltpu

H, DK, DV = 8, 16, 16


def _post_body(x_ref, q_ref, wfc_ref, g_ref, b_ref, o_ref):
    x = jnp.dot(x_ref[...], wfc_ref[...], preferred_element_type=jnp.float32)
    x = x + q_ref[...]
    mean = jnp.mean(x, axis=-1, keepdims=True)
    var = jnp.mean((x - mean) ** 2, axis=-1, keepdims=True)
    o_ref[...] = g_ref[...] * (x - mean) * jax.lax.rsqrt(var + 1e-5) + b_ref[...]


def _post(xdiv, input_q, wfc, gamma, beta):
    R, D = xdiv.shape
    BR = 1000
    grid = (R // BR,)
    return pl.pallas_call(
        _post_body,
        grid=grid,
        in_specs=[
            pl.BlockSpec((BR, D), lambda i: (i, 0)),
            pl.BlockSpec((BR, D), lambda i: (i, 0)),
            pl.BlockSpec((D, D), lambda i: (0, 0)),
            pl.BlockSpec((1, D), lambda i: (0, 0)),
            pl.BlockSpec((1, D), lambda i: (0, 0)),
        ],
        out_specs=pl.BlockSpec((BR, D), lambda i: (i, 0)),
        out_shape=jax.ShapeDtypeStruct((R, D), jnp.float32),
    )(xdiv, input_q, wfc, gamma.reshape(1, D), beta.reshape(1, D))


def kernel(edge_indices, edge_features, input_Q, input_K, input_V,
           Wq, Wk, Wv, We, Wfc, gamma, beta):
    B, N, D = input_Q.shape
    E = edge_indices.shape[-1]

    Q = (input_Q @ Wq).reshape(B, N, H, DK)
    K = (input_K @ Wk).reshape(B, N, H, DK)
    V = (input_V @ Wv).reshape(B, N, H, DV)
    Ep = (edge_features @ We).reshape(B, E, H, DV)

    src = edge_indices[:, 0]
    tgt = edge_indices[:, 1]

    def per_batch(Qb, Kb, Vb, Eb, sb, tb):
        Qe = Qb[sb]
        Ke = Kb[tb]
        Ve = Vb[tb]
        scores = jnp.sum((Qe * Ke) * Eb, axis=-1) * (1.0 / np.sqrt(DK))
        attn = jnp.exp(jnp.clip(scores, -5.0, 5.0))
        msg = attn[..., None] * Ve
        out = jax.ops.segment_sum(msg, sb, num_segments=N)
        coeff = jax.ops.segment_sum(attn, sb, num_segments=N)
        return out / (coeff[..., None] + 1e-8), attn

    out, attn = jax.vmap(per_batch)(Q, K, V, Ep, src, tgt)
    xdiv = out.reshape(B * N, H * DV)
    res = _post(xdiv, input_Q.reshape(B * N, D), Wfc, gamma, beta)
    res = res.reshape(B, N, D)
    attn_out = attn[-1:].transpose(0, 2, 1)[..., None]
    return res, attn_out


# speedup vs baseline: 9.3754x; 9.3712x over previous
import functools

import jax
import jax.numpy as jnp
from jax import lax
from jax.experimental import pallas as pl
from jax.experimental.pallas import tpu as pltpu
from jax.experimental.pallas import tpu_sc as plsc

B, N, E, D = 2, 10000, 160000, 128
H, DK, DV = 8, 16, 16
L = 16
NS = 16
C = 32
NCHUNKS = E // C
NCT_BASE = NCHUNKS // NS
NCT_REM = NCHUNKS % NS
NP = 10240
RPT = NP // NS
NPC = NP // L
CPT = NPC // NS


def _mm_body(x_ref, w_ref, o_ref):
    o_ref[...] = jnp.dot(x_ref[...], w_ref[...],
                         preferred_element_type=jnp.float32)


def _mm(x, w, br):
    R, Din = x.shape
    Dout = w.shape[1]
    return pl.pallas_call(
        _mm_body,
        grid=(R // br,),
        in_specs=[
            pl.BlockSpec((br, Din), lambda i: (i, 0)),
            pl.BlockSpec((Din, Dout), lambda i: (0, 0)),
        ],
        out_specs=pl.BlockSpec((br, Dout), lambda i: (i, 0)),
        out_shape=jax.ShapeDtypeStruct((R, Dout), jnp.float32),
    )(x, w)


def _post_body(acc_ref, c_ref, q_ref, wfc_ref, g_ref, b_ref, o_ref):
    i0 = lax.broadcasted_iota(jnp.int32, (H, D), 0)
    i1 = lax.broadcasted_iota(jnp.int32, (H, D), 1)
    expand = (i1 // DV == i0).astype(jnp.float32)
    scale = jnp.dot(c_ref[...], expand, preferred_element_type=jnp.float32)
    xdiv = acc_ref[...] / (scale + 1e-8)
    x = jnp.dot(xdiv, wfc_ref[...], preferred_element_type=jnp.float32)
    x = x + q_ref[...]
    mean = jnp.mean(x, axis=-1, keepdims=True)
    var = jnp.mean((x - mean) ** 2, axis=-1, keepdims=True)
    o_ref[...] = g_ref[...] * (x - mean) * lax.rsqrt(var + 1e-5) + b_ref[...]


def _post(acc, coeff, input_q, wfc, gamma, beta):
    R = acc.shape[0]
    BR = 1000
    return pl.pallas_call(
        _post_body,
        grid=(R // BR,),
        in_specs=[
            pl.BlockSpec((BR, D), lambda i: (i, 0)),
            pl.BlockSpec((BR, H), lambda i: (i, 0)),
            pl.BlockSpec((BR, D), lambda i: (i, 0)),
            pl.BlockSpec((D, D), lambda i: (0, 0)),
            pl.BlockSpec((1, D), lambda i: (0, 0)),
            pl.BlockSpec((1, D), lambda i: (0, 0)),
        ],
        out_specs=pl.BlockSpec((BR, D), lambda i: (i, 0)),
        out_shape=jax.ShapeDtypeStruct((R, D), jnp.float32),
    )(acc, coeff, input_q, wfc, gamma.reshape(1, D), beta.reshape(1, D))


def _sc_body(qf, kf, vf, epf, ei,
             outacc, coeffout, attn_out,
             out_sh, cf_sh,
             srcv, qidx, tidx, crid, cbase,
             qrows, krows, vrows, eprows,
             msg, msgc, attv, sem):
    b = lax.axis_index("c")
    t = lax.axis_index("s")
    r0 = t * RPT
    bNP = b * NP

    zv = jnp.zeros((L,), jnp.float32)

    def _zc(i, carry):
        for c in range(D // L):
            msg[i, pl.ds(c * L, L)] = zv
            msgc[i, pl.ds(c * L, L)] = zv
        return carry

    lax.fori_loop(0, C, _zc, None)

    for j in range(RPT // C):
        pltpu.sync_copy(msg, out_sh.at[pl.ds(r0 + j * C, C)])
    pltpu.sync_copy(msg, cf_sh.at[pl.ds(t * CPT, C)])
    pltpu.sync_copy(msg.at[pl.ds(0, CPT - C)],
                    cf_sh.at[pl.ds(t * CPT + C, CPT - C)])
    plsc.subcore_barrier()

    rowi = lax.iota(jnp.int32, L)

    def chunk_body(ci, carry):
        eb = (ci * NS + t) * C
        g0 = b * E + eb
        pltpu.sync_copy(ei.at[pl.ds(g0, C)], srcv)
        pltpu.sync_copy(ei.at[pl.ds(B * E + g0, C)], qidx)
        pltpu.sync_copy(ei.at[pl.ds(2 * B * E + g0, C)], tidx)
        pltpu.sync_copy(ei.at[pl.ds(3 * B * E + g0, C)], crid)
        pltpu.sync_copy(ei.at[pl.ds(4 * B * E + g0, C)], cbase)
        pltpu.async_copy(qf.at[qidx], qrows, sem).wait()
        pltpu.async_copy(kf.at[tidx], krows, sem).wait()
        pltpu.async_copy(vf.at[tidx], vrows, sem).wait()
        pltpu.async_copy(epf.at[pl.ds(g0, C)], eprows, sem).wait()

        def head_body(h, hcarry):
            def group_body(g, gcarry):
                rbase = rowi + g * L
                cb = cbase[pl.ds(g * L, L)] + h
                acc = None
                for k in range(L):
                    colv = jnp.full((L,), h * L + k, jnp.int32)
                    qc = plsc.load_gather(qrows, [rbase, colv])
                    kc = plsc.load_gather(krows, [rbase, colv])
                    ec = plsc.load_gather(eprows, [rbase, colv])
                    p = qc * kc * ec
                    acc = p if acc is None else acc + p
                a = jnp.exp(jnp.clip(acc * 0.25, -5.0, 5.0))
                plsc.store_scatter(attv, [h * C + g * L + rowi], a)
                plsc.store_scatter(msgc, [rbase, cb], a)
                for k in range(L):
                    colv = jnp.full((L,), h * L + k, jnp.int32)
                    vc = plsc.load_gather(vrows, [rbase, colv])
                    plsc.store_scatter(msg, [rbase, colv], a * vc)
                return gcarry

            return lax.fori_loop(0, C // L, group_body, hcarry)

        lax.fori_loop(0, H, head_body, None)

        pltpu.sync_copy(msg, out_sh.at[srcv], add=True)
        pltpu.sync_copy(msgc, cf_sh.at[crid], add=True)

        def zero_heads(h, hcarry):
            def zero_group(g, gcarry):
                rbase = rowi + g * L
                cb = cbase[pl.ds(g * L, L)] + h
                plsc.store_scatter(msgc, [rbase, cb], zv)
                return gcarry

            return lax.fori_loop(0, C // L, zero_group, hcarry)

        lax.fori_loop(0, H, zero_heads, None)

        for h in range(H):
            pltpu.sync_copy(attv.at[pl.ds(h * C, C)],
                            attn_out.at[pl.ds(b * H * E + h * E + eb, C)])
        return carry

    nct = NCT_BASE + (t < NCT_REM).astype(jnp.int32)
    lax.fori_loop(0, nct, chunk_body, None)
    plsc.subcore_barrier()

    for j in range(RPT // C):
        rr = r0 + j * C
        pltpu.sync_copy(out_sh.at[pl.ds(rr, C)], msg)
        pltpu.sync_copy(msg, outacc.at[pl.ds(bNP + rr, C)])
    cc = t * CPT
    pltpu.sync_copy(cf_sh.at[pl.ds(cc, C)], msg)
    pltpu.sync_copy(msg, coeffout.at[pl.ds(b * NPC + cc, C)])
    pltpu.sync_copy(cf_sh.at[pl.ds(cc + C, CPT - C)], msg.at[pl.ds(0, CPT - C)])
    pltpu.sync_copy(msg.at[pl.ds(0, CPT - C)],
                    coeffout.at[pl.ds(b * NPC + cc + C, CPT - C)])


_SC_KW = dict(
    out_type=(
        jax.ShapeDtypeStruct((B * NP, D), jnp.float32),
        jax.ShapeDtypeStruct((B * NPC, D), jnp.float32),
        jax.ShapeDtypeStruct((B * H * E,), jnp.float32),
    ),
    mesh=plsc.VectorSubcoreMesh(core_axis_name="c", subcore_axis_name="s",
                                num_cores=2, num_subcores=NS),
    scratch_types=[
        pltpu.VMEM_SHARED((NP, D), jnp.float32),
        pltpu.VMEM_SHARED((NPC, D), jnp.float32),
        pltpu.VMEM((C,), jnp.int32),
        pltpu.VMEM((C,), jnp.int32),
        pltpu.VMEM((C,), jnp.int32),
        pltpu.VMEM((C,), jnp.int32),
        pltpu.VMEM((C,), jnp.int32),
        pltpu.VMEM((C, D), jnp.float32),
        pltpu.VMEM((C, D), jnp.float32),
        pltpu.VMEM((C, D), jnp.float32),
        pltpu.VMEM((C, D), jnp.float32),
        pltpu.VMEM((C, D), jnp.float32),
        pltpu.VMEM((C, D), jnp.float32),
        pltpu.VMEM((H * C,), jnp.float32),
        pltpu.SemaphoreType.DMA,
    ],
    compiler_params=pltpu.CompilerParams(needs_layout_passes=False),
)

_sc_edge = functools.partial(pl.kernel, **_SC_KW)(_sc_body)


def kernel(edge_indices, edge_features, input_Q, input_K, input_V,
           Wq, Wk, Wv, We, Wfc, gamma, beta):
    qf = _mm(input_Q.reshape(B * N, D), Wq, 800)
    kf = _mm(input_K.reshape(B * N, D), Wk, 800)
    vf = _mm(input_V.reshape(B * N, D), Wv, 800)
    epf = _mm(edge_features.reshape(B * E, D), We, 800)
    ei32 = edge_indices.astype(jnp.int32)
    boff = (jnp.arange(B, dtype=jnp.int32) * N)[:, None]
    src_raw = ei32[:, 0]
    ei_flat = jnp.concatenate(
        [src_raw.reshape(-1), (src_raw + boff).reshape(-1),
         (ei32[:, 1] + boff).reshape(-1),
         (src_raw >> 4).reshape(-1), ((src_raw & 15) * 8).reshape(-1)])

    outacc, coeffp, attn_flat = _sc_edge(qf, kf, vf, epf, ei_flat)
    outacc = outacc.reshape(B, NP, D)[:, :N].reshape(B * N, D)
    coeff = coeffp.reshape(B, NP, H)[:, :N].reshape(B * N, H)

    res = _post(outacc, coeff, input_Q.reshape(B * N, D), Wfc, gamma, beta)
    res = res.reshape(B, N, D)
    attn_out = attn_flat.reshape(B, H, E)[-1:][..., None]
    return res, attn_out
